# Initial kernel scaffold; baseline (speedup 1.0000x reference)
import jax
import jax.numpy as jnp
from jax.experimental import pallas as pl


def kernel(feature_data, edge_info, W_pre1, b_pre1, W_pre2, b_pre2, W_pre3, b_pre3, Wl1, bl1, Wr1, Wl2, bl2, Wr2, Wl3, bl3, Wr3, W_post1, b_post1, W_post2, b_post2, W_post3, b_post3):
    raise NotImplementedError("write your pallas kernel here")


# speedup vs baseline: 6.1433x; 6.1433x over previous
import functools

import jax
import jax.numpy as jnp
from jax import lax
from jax.experimental import pallas as pl
from jax.experimental.pallas import tpu as pltpu
from jax.experimental.pallas import tpu_sc as plsc

N = 10000
E = 320000
D = 128

ROWS = 2000
NB = N // ROWS
NSC = 2
NTILES = 16
CHUNK = 128
GCH = 16
GROUPS = 10
PER_TILE = GROUPS * GCH * CHUNK
E_PAD = NTILES * PER_TILE
ACC_ROWS = N + 16
STRIPE = 624


def _matT(x, W):
    return lax.dot_general(x, W, (((1,), (1,)), ((), ())),
                           preferred_element_type=jnp.float32)


def _pre_body(x_ref, w1, b1, w2, b2, w3, b3, h_ref, cmax_ref):
    i = pl.program_id(0)
    h = jnp.maximum(_matT(x_ref[...], w1[...]) + b1[...], 0.0)
    h = jnp.maximum(_matT(h, w2[...]) + b2[...], 0.0)
    h = jnp.maximum(_matT(h, w3[...]) + b3[...], 0.0)
    h_ref[...] = h
    m8 = jnp.broadcast_to(jnp.max(h, axis=0, keepdims=True), (8, D))

    @pl.when(i == 0)
    def _():
        cmax_ref[...] = m8

    @pl.when(i > 0)
    def _():
        cmax_ref[...] = jnp.maximum(cmax_ref[...], m8)


def _pre(x, w1, b1, w2, b2, w3, b3):
    full = pl.BlockSpec((D, D), lambda i: (0, 0))
    bias = pl.BlockSpec((1, D), lambda i: (0, 0))
    return pl.pallas_call(
        _pre_body,
        grid=(NB,),
        in_specs=[pl.BlockSpec((ROWS, D), lambda i: (i, 0)),
                  full, bias, full, bias, full, bias],
        out_specs=[pl.BlockSpec((ROWS, D), lambda i: (i, 0)),
                   pl.BlockSpec((8, D), lambda i: (0, 0))],
        out_shape=[jax.ShapeDtypeStruct((N, D), jnp.float32),
                   jax.ShapeDtypeStruct((8, D), jnp.float32)],
    )(x, w1, b1, w2, b2, w3, b3)


def _pq_body(h_ref, cmax_ref, t_ref):
    p = pl.program_id(0)
    h = h_ref[...]
    e = jnp.exp(h - cmax_ref[0:1, :])
    t_ref[...] = jnp.where(p == 0, e, e * h)


def _pq(h, cmax):
    return pl.pallas_call(
        _pq_body,
        grid=(2, NB),
        in_specs=[pl.BlockSpec((ROWS, D), lambda p, i: (i, 0)),
                  pl.BlockSpec((8, D), lambda p, i: (0, 0))],
        out_specs=pl.BlockSpec((ROWS, D), lambda p, i: (p * NB + i, 0)),
        out_shape=jax.ShapeDtypeStruct((2 * N, D), jnp.float32),
    )(h, cmax)


def _seg_sums(T, src4, dst3, zeros):
    mesh = plsc.VectorSubcoreMesh(core_axis_name="c", subcore_axis_name="s")

    @functools.partial(
        pl.kernel,
        out_type=jax.ShapeDtypeStruct((NSC, N, D), jnp.float32),
        mesh=mesh,
        scratch_types=[
            pltpu.VMEM_SHARED((ACC_ROWS, D), jnp.float32),
            pltpu.VMEM((GCH, CHUNK), jnp.int32),
            pltpu.VMEM((GCH, CHUNK), jnp.int32),
            pltpu.VMEM((CHUNK, D), jnp.float32),
        ],
    )
    def k(t_hbm, src_hbm, dst_hbm, z_hbm, out_hbm, accum, srcb, dstb, rows):
        c = lax.axis_index("c")
        s = lax.axis_index("s")

        @pl.when(s < NTILES - 1)
        def _():
            pltpu.sync_copy(z_hbm.at[pl.ds(0, STRIPE)],
                            accum.at[pl.ds(s * STRIPE, STRIPE)])

        @pl.when(s == NTILES - 1)
        def _():
            pltpu.sync_copy(z_hbm,
                            accum.at[pl.ds((NTILES - 1) * STRIPE,
                                           ACC_ROWS - (NTILES - 1) * STRIPE)])

        plsc.subcore_barrier()

        @pl.loop(0, GROUPS)
        def _(g):
            pltpu.sync_copy(src_hbm.at[c, s, pl.ds(g * GCH, GCH)], srcb)
            pltpu.sync_copy(dst_hbm.at[s, pl.ds(g * GCH, GCH)], dstb)

            @pl.loop(0, GCH)
            def _(j):
                pltpu.sync_copy(t_hbm.at[srcb.at[j]], rows)
                pltpu.sync_copy(rows, accum.at[dstb.at[j]], add=True)

        plsc.subcore_barrier()

        @pl.when(s < NTILES - 1)
        def _():
            pltpu.sync_copy(accum.at[pl.ds(s * STRIPE, STRIPE)],
                            out_hbm.at[c, pl.ds(s * STRIPE, STRIPE)])

        @pl.when(s == NTILES - 1)
        def _():
            tail0 = (NTILES - 1) * STRIPE
            pltpu.sync_copy(accum.at[pl.ds(tail0, N - tail0)],
                            out_hbm.at[c, pl.ds(tail0, N - tail0)])

    return k(T, src4, dst3, zeros)


def _combine_body(den_ref, num_ref, h_ref, wl, bl, wr, y_ref, cmax_ref):
    i = pl.program_id(0)
    aggr = num_ref[0] / (den_ref[0] + 1e-16)
    y = _matT(aggr, wl[...]) + bl[...] + _matT(h_ref[...], wr[...])
    y = jnp.maximum(y, 0.0)
    y_ref[...] = y
    m8 = jnp.broadcast_to(jnp.max(y, axis=0, keepdims=True), (8, D))

    @pl.when(i == 0)
    def _():
        cmax_ref[...] = m8

    @pl.when(i > 0)
    def _():
        cmax_ref[...] = jnp.maximum(cmax_ref[...], m8)


def _combine(sums, h, wl, bl, wr):
    full = pl.BlockSpec((D, D), lambda i: (0, 0))
    bias = pl.BlockSpec((1, D), lambda i: (0, 0))
    return pl.pallas_call(
        _combine_body,
        grid=(NB,),
        in_specs=[pl.BlockSpec((1, ROWS, D), lambda i: (0, i, 0)),
                  pl.BlockSpec((1, ROWS, D), lambda i: (1, i, 0)),
                  pl.BlockSpec((ROWS, D), lambda i: (i, 0)),
                  full, bias, full],
        out_specs=[pl.BlockSpec((ROWS, D), lambda i: (i, 0)),
                   pl.BlockSpec((8, D), lambda i: (0, 0))],
        out_shape=[jax.ShapeDtypeStruct((N, D), jnp.float32),
                   jax.ShapeDtypeStruct((8, D), jnp.float32)],
    )(sums, sums, h, wl, bl, wr)


def _final_body(den_ref, num_ref, h_ref, wl, bl, wr,
                w1, b1, w2, b2, w3, b3, o_ref):
    aggr = num_ref[0] / (den_ref[0] + 1e-16)
    y = _matT(aggr, wl[...]) + bl[...] + _matT(h_ref[...], wr[...])
    y = jnp.maximum(y, 0.0)
    y = jnp.maximum(_matT(y, w1[...]) + b1[...], 0.0)
    y = jnp.maximum(_matT(y, w2[...]) + b2[...], 0.0)
    o_ref[...] = jnp.tanh(_matT(y, w3[...]) + b3[...])


def _final(sums, h, wl, bl, wr, w1, b1, w2, b2, w3, b3):
    full = pl.BlockSpec((D, D), lambda i: (0, 0))
    bias = pl.BlockSpec((1, D), lambda i: (0, 0))
    return pl.pallas_call(
        _final_body,
        grid=(NB,),
        in_specs=[pl.BlockSpec((1, ROWS, D), lambda i: (0, i, 0)),
                  pl.BlockSpec((1, ROWS, D), lambda i: (1, i, 0)),
                  pl.BlockSpec((ROWS, D), lambda i: (i, 0)),
                  full, bias, full, full, bias, full, bias, full, bias],
        out_specs=pl.BlockSpec((ROWS, D), lambda i: (i, 0)),
        out_shape=jax.ShapeDtypeStruct((N, D), jnp.float32),
    )(sums, sums, h, wl, bl, wr, w1, b1, w2, b2, w3, b3)


def kernel(feature_data, edge_info,
           W_pre1, b_pre1, W_pre2, b_pre2, W_pre3, b_pre3,
           Wl1, bl1, Wr1, Wl2, bl2, Wr2, Wl3, bl3, Wr3,
           W_post1, b_post1, W_post2, b_post2, W_post3, b_post3):
    def r(b):
        return b.reshape(1, D)

    npad = E_PAD - E
    src = jnp.concatenate([edge_info[0], jnp.zeros((npad,), jnp.int32)])
    dst = jnp.concatenate([edge_info[1], jnp.full((npad,), N, jnp.int32)])
    src4 = jnp.stack([src, src + N]).reshape(NSC, NTILES, GROUPS * GCH, CHUNK)
    dst3 = dst.reshape(NTILES, GROUPS * GCH, CHUNK)
    zeros = jnp.zeros((ACC_ROWS - (NTILES - 1) * STRIPE, D), jnp.float32)

    h, cmax = _pre(feature_data, W_pre1, r(b_pre1),
                   W_pre2, r(b_pre2), W_pre3, r(b_pre3))

    sage = [(Wl1, bl1, Wr1), (Wl2, bl2, Wr2), (Wl3, bl3, Wr3)]
    for layer, (wl, bl, wr) in enumerate(sage):
        T = _pq(h, cmax)
        sums = _seg_sums(T, src4, dst3, zeros)
        if layer < 2:
            h, cmax = _combine(sums, h, wl, r(bl), wr)
        else:
            out = _final(sums, h, wl, r(bl), wr,
                         W_post1, r(b_post1), W_post2, r(b_post2),
                         W_post3, r(b_post3))
    return out
